# baseline (speedup 1.0000x reference)
import functools

import jax
import jax.numpy as jnp
from jax import lax
from jax.experimental import pallas as pl
from jax.experimental.pallas import tpu as pltpu
from jax.experimental.pallas import tpu_sc as plsc

_N, _F, _T = 131072, 64, 8
_FT = _F * _T
_NC, _NS, _L = 2, 16, 16
_NW = _NC * _NS
_RPW = _N // _NW
_R = 64
_G = _RPW // _R
_XC = _R * _F
_OC = _R * _FT
_J = _FT // _L


def _sc_body(x_hbm, thr_hbm, out_hbm, thr_v, xb0, xb1, ob0, ob1,
             si0, si1, so0, so1):
    wid = lax.axis_index("s") * _NC + lax.axis_index("c")
    base = wid * _RPW

    pltpu.sync_copy(thr_hbm, thr_v)

    lane = lax.iota(jnp.int32, _L)
    hi8 = lax.shift_right_logical(lane, jnp.full((_L,), 3, jnp.int32))
    one = jnp.full((_L,), 1.0, jnp.float32)
    zero = jnp.full((_L,), 0.0, jnp.float32)
    thrs = [thr_v[pl.ds(_L * j, _L)] for j in range(_J)]
    gidxs = [hi8 + jnp.full((_L,), 2 * m, jnp.int32) for m in range(8)]

    xbs, obs, sis, sos = (xb0, xb1), (ob0, ob1), (si0, si1), (so0, so1)

    def start_in(g, b):
        pltpu.async_copy(
            x_hbm.at[pl.ds((base + g * _R) * _F, _XC)], xbs[b], sis[b])

    def wait_in(b):
        pltpu.make_async_copy(
            x_hbm.at[pl.ds(0, _XC)], xbs[b], sis[b]).wait()

    def start_out(g, b):
        pltpu.async_copy(
            obs[b], out_hbm.at[pl.ds((base + g * _R) * _FT, _OC)], sos[b])

    def wait_out(b):
        pltpu.make_async_copy(
            obs[b], out_hbm.at[pl.ds(0, _OC)], sos[b]).wait()

    def _vgather(vec, idx):
        return lax.gather(
            vec, idx[:, None],
            lax.GatherDimensionNumbers(
                offset_dims=(), collapsed_slice_dims=(0,),
                start_index_map=(0,)),
            (1,), mode=lax.GatherScatterMode.PROMISE_IN_BOUNDS)

    def compute(xb, ob):
        @plsc.parallel_loop(0, _R, unroll=1)
        def row(r):
            rb = r * _F
            outb = r * _FT
            for k in range(_F // _L):
                xv = xb[pl.ds(rb + _L * k, _L)]
                for m in range(8):
                    j = 8 * k + m
                    xg = _vgather(xv, gidxs[m])
                    ob[pl.ds(outb + _L * j, _L)] = jnp.where(
                        xg > thrs[j], one, zero)

    start_in(0, 0)
    start_in(1, 1)

    def pair(g2, c):
        for b in range(2):
            g = 2 * g2 + b
            wait_in(b)

            @pl.when(g2 > 0)
            def _():
                wait_out(b)

            compute(xbs[b], obs[b])
            start_out(g, b)

            @pl.when(g2 < _G // 2 - 1)
            def _():
                start_in(g + 2, b)
        return c

    lax.fori_loop(0, _G // 2, pair, 0)
    wait_out(0)
    wait_out(1)


@functools.partial(jax.jit, static_argnames=())
def kernel(x, thresholds):
    n, f = x.shape
    t = thresholds.shape[-1]
    mesh = plsc.VectorSubcoreMesh(
        core_axis_name="c", subcore_axis_name="s",
        num_cores=_NC, num_subcores=_NS)
    out_flat = pl.kernel(
        _sc_body,
        out_type=jax.ShapeDtypeStruct((n * f * t,), jnp.float32),
        mesh=mesh,
        compiler_params=pltpu.CompilerParams(needs_layout_passes=False),
        scratch_types=[
            pltpu.VMEM((_FT,), jnp.float32),
            pltpu.VMEM((_XC,), jnp.float32),
            pltpu.VMEM((_XC,), jnp.float32),
            pltpu.VMEM((_OC,), jnp.float32),
            pltpu.VMEM((_OC,), jnp.float32),
            pltpu.SemaphoreType.DMA,
            pltpu.SemaphoreType.DMA,
            pltpu.SemaphoreType.DMA,
            pltpu.SemaphoreType.DMA,
        ],
    )(x.reshape(-1), thresholds.reshape(-1))
    return out_flat.reshape(n, f * t)


# speedup vs baseline: 1.7972x; 1.7972x over previous
import functools

import jax
import jax.numpy as jnp
import numpy as np
from jax.experimental import pallas as pl
from jax.experimental.pallas import tpu as pltpu

_N, _F, _T = 131072, 64, 8
_BN = 4096


def _body(x_ref, thr_ref, s_ref, o_ref):
    xr = jax.lax.dot_general(
        x_ref[...], s_ref[...],
        dimension_numbers=(((1,), (0,)), ((), ())),
        precision=jax.lax.Precision.HIGHEST,
        preferred_element_type=jnp.float32,
    )
    o_ref[...] = (xr > thr_ref[...]).astype(jnp.float32)


@functools.partial(jax.jit, static_argnames=())
def kernel(x, thresholds):
    n, f = x.shape
    t = thresholds.shape[-1]
    thr_flat = thresholds.reshape(1, f * t)
    sel = jnp.asarray(np.repeat(np.eye(f, dtype=np.float32), t, axis=1))
    grid = (n // _BN,)
    out = pl.pallas_call(
        _body,
        grid=grid,
        in_specs=[
            pl.BlockSpec((_BN, f), lambda i: (i, 0)),
            pl.BlockSpec((1, f * t), lambda i: (0, 0)),
            pl.BlockSpec((f, f * t), lambda i: (0, 0)),
        ],
        out_specs=pl.BlockSpec((_BN, f * t), lambda i: (i, 0)),
        out_shape=jax.ShapeDtypeStruct((n, f * t), jnp.float32),
        compiler_params=pltpu.CompilerParams(
            dimension_semantics=("parallel",),
        ),
    )(x, thr_flat, sel)
    return out
